# Initial kernel scaffold; baseline (speedup 1.0000x reference)
import jax
import jax.numpy as jnp
from jax.experimental import pallas as pl


def kernel(source_points, source_features, target_points):
    raise NotImplementedError("write your pallas kernel here")


# speedup vs baseline: 8.0249x; 8.0249x over previous
import functools

import jax
import jax.numpy as jnp
from jax import lax
from jax.experimental import pallas as pl
from jax.experimental.pallas import tpu as pltpu
from jax.experimental.pallas import tpu_sc as plsc

M = 4096
N = 4096
C = 256
KNN = 8
TBLK = 256
GRID = M // TBLK


def _tc_topk_body(tgt_ref, spT_ref, idx_ref, w_ref):
    tb = tgt_ref[...]
    acc = jnp.zeros((TBLK, N), jnp.float32)
    for d in range(3):
        diff = tb[:, d:d + 1] - spT_ref[d:d + 1, :]
        acc = acc + diff * diff
    iota = lax.broadcasted_iota(jnp.int32, (TBLK, N), 1)
    cur = acc
    idxs, vals = [], []
    for _ in range(KNN):
        mv = jnp.min(cur, axis=1, keepdims=True)
        cand = jnp.where(cur == mv, iota, jnp.int32(N))
        mi = jnp.min(cand, axis=1, keepdims=True)
        idxs.append(mi)
        vals.append(mv)
        cur = jnp.where(iota == mi, jnp.float32(jnp.inf), cur)
    knn_d = jnp.concatenate(vals, axis=1)
    knn_i = jnp.concatenate(idxs, axis=1)
    w = 1.0 / (knn_d + 1e-8)
    w = w / jnp.sum(w, axis=1, keepdims=True)
    idx_ref[...] = knn_i
    w_ref[...] = w


def _topk_weights(target_points, spT8):
    return pl.pallas_call(
        _tc_topk_body,
        grid=(GRID,),
        in_specs=[
            pl.BlockSpec((TBLK, 3), lambda i: (i, 0)),
            pl.BlockSpec((8, N), lambda i: (0, 0)),
        ],
        out_specs=[
            pl.BlockSpec((TBLK, KNN), lambda i: (i, 0)),
            pl.BlockSpec((TBLK, KNN), lambda i: (i, 0)),
        ],
        out_shape=[
            jax.ShapeDtypeStruct((M, KNN), jnp.int32),
            jax.ShapeDtypeStruct((M, KNN), jnp.float32),
        ],
    )(target_points, spT8)


NUM_SC = 2
NUM_SUBCORES = 16
NW = NUM_SC * NUM_SUBCORES
TPW = M // NW
RPC = 128
TPC = RPC // KNN
NCH = TPW // TPC
LANES = 16


def _sc_gather(feat, idx_flat, wexp):
    mesh = plsc.VectorSubcoreMesh(core_axis_name="c", subcore_axis_name="s")

    @functools.partial(
        pl.kernel,
        mesh=mesh,
        out_type=jax.ShapeDtypeStruct((M, C), jnp.float32),
        scratch_types=[
            pltpu.VMEM((RPC,), jnp.int32),
            pltpu.VMEM((RPC, C), jnp.float32),
            pltpu.VMEM((RPC, LANES), jnp.float32),
            pltpu.VMEM((TPC, C), jnp.float32),
            pltpu.SemaphoreType.DMA,
        ],
    )
    def k(feat_hbm, idxf_hbm, wexp_hbm, out_hbm, idx_v, rows_v, wexp_v,
          outb_v, sem):
        cid = lax.axis_index("c")
        sid = lax.axis_index("s")
        wid = sid * NUM_SC + cid
        for ch in range(NCH):
            row0 = wid * (TPW * KNN) + ch * RPC
            pltpu.sync_copy(idxf_hbm.at[pl.ds(row0, RPC)], idx_v)
            cp = pltpu.async_copy(feat_hbm.at[idx_v], rows_v, sem)
            pltpu.sync_copy(wexp_hbm.at[pl.ds(row0, RPC), :], wexp_v)
            cp.wait()

            def t_body(t, carry):
                for cc in range(C // LANES):
                    acc = jnp.zeros((LANES,), jnp.float32)
                    for j in range(KNN):
                        r = t * KNN + j
                        acc = acc + wexp_v[r, :] * rows_v[r, pl.ds(cc * LANES, LANES)]
                    outb_v[t, pl.ds(cc * LANES, LANES)] = acc
                return carry

            lax.fori_loop(0, TPC, t_body, 0)
            pltpu.sync_copy(outb_v, out_hbm.at[pl.ds(wid * TPW + ch * TPC, TPC), :])

    return k(feat, idx_flat, wexp)


def kernel(source_points, source_features, target_points):
    spT8 = jnp.zeros((8, N), jnp.float32).at[:3, :].set(source_points.T)
    knn_i, w = _topk_weights(target_points, spT8)
    idx_flat = knn_i.reshape(M * KNN)
    wexp = jnp.broadcast_to(w.reshape(M * KNN, 1), (M * KNN, LANES))
    return _sc_gather(source_features, idx_flat, wexp)


# speedup vs baseline: 10.5334x; 1.3126x over previous
import functools

import jax
import jax.numpy as jnp
from jax import lax
from jax.experimental import pallas as pl
from jax.experimental.pallas import tpu as pltpu
from jax.experimental.pallas import tpu_sc as plsc

M = 4096
N = 4096
C = 256
KNN = 8
TBLK = 256
GRID = M // TBLK


NCHK = 8
CW = N // NCHK


def _tc_topk_body(tgt_ref, spT_ref, idx_ref, w_ref):
    tb = tgt_ref[...]
    dist = jnp.zeros((TBLK, N), jnp.float32)
    for d in range(3):
        diff = tb[:, d:d + 1] - spT_ref[d:d + 1, :]
        dist = dist + diff * diff
    iota = lax.broadcasted_iota(jnp.int32, (TBLK, N), 1)
    bits = lax.bitcast_convert_type(dist + 1.0, jnp.int32)
    tags = lax.shift_right_logical(iota, 9)
    T = lax.bitcast_convert_type((bits & (-8)) | tags, jnp.float32)
    iota512f = lax.broadcasted_iota(jnp.int32, (TBLK, CW), 1).astype(jnp.float32)
    vals, gidxs = [], []
    for p in range(KNN):
        f = T[:, 0:CW]
        for c in range(1, NCHK):
            f = jnp.minimum(f, T[:, c * CW:(c + 1) * CW])
        mf = jnp.min(f, axis=1, keepdims=True)
        cand = jnp.where(f == mf, iota512f, jnp.float32(1e9))
        mi = jnp.min(cand, axis=1, keepdims=True)
        mfb = lax.bitcast_convert_type(mf, jnp.int32)
        cstar = mfb & 7
        gidx = cstar * CW + mi.astype(jnp.int32)
        dval = lax.bitcast_convert_type(mfb & (-8), jnp.float32) - 1.0
        vals.append(dval)
        gidxs.append(gidx)
        if p < KNN - 1:
            T = jnp.where(iota == gidx, jnp.float32(jnp.inf), T)
    knn_d = jnp.concatenate(vals, axis=1)
    knn_i = jnp.concatenate(gidxs, axis=1)
    w = 1.0 / (knn_d + 1e-8)
    w = w / jnp.sum(w, axis=1, keepdims=True)
    idx_ref[...] = knn_i
    w_ref[...] = w


def _topk_weights(tgt8, spT8):
    return pl.pallas_call(
        _tc_topk_body,
        grid=(GRID,),
        in_specs=[
            pl.BlockSpec((TBLK, 8), lambda i: (i, 0)),
            pl.BlockSpec((8, N), lambda i: (0, 0)),
        ],
        out_specs=[
            pl.BlockSpec((TBLK, KNN), lambda i: (i, 0)),
            pl.BlockSpec((TBLK, KNN), lambda i: (i, 0)),
        ],
        out_shape=[
            jax.ShapeDtypeStruct((M, KNN), jnp.int32),
            jax.ShapeDtypeStruct((M, KNN), jnp.float32),
        ],
    )(tgt8, spT8)


NUM_SC = 2
NUM_SUBCORES = 16
NW = NUM_SC * NUM_SUBCORES
TPW = M // NW
RPC = 128
TPC = RPC // KNN
NCH = TPW // TPC
LANES = 16


def _sc_gather(feat, idx_flat, wexp):
    mesh = plsc.VectorSubcoreMesh(core_axis_name="c", subcore_axis_name="s")

    @functools.partial(
        pl.kernel,
        mesh=mesh,
        out_type=jax.ShapeDtypeStruct((M, C), jnp.float32),
        scratch_types=[
            pltpu.VMEM((RPC,), jnp.int32),
            pltpu.VMEM((RPC, C), jnp.float32),
            pltpu.VMEM((RPC, LANES), jnp.float32),
            pltpu.VMEM((TPC, C), jnp.float32),
            pltpu.SemaphoreType.DMA,
        ],
    )
    def k(feat_hbm, idxf_hbm, wexp_hbm, out_hbm, idx_v, rows_v, wexp_v,
          outb_v, sem):
        cid = lax.axis_index("c")
        sid = lax.axis_index("s")
        wid = sid * NUM_SC + cid
        for ch in range(NCH):
            row0 = wid * (TPW * KNN) + ch * RPC
            pltpu.sync_copy(idxf_hbm.at[pl.ds(row0, RPC)], idx_v)
            cp = pltpu.async_copy(feat_hbm.at[idx_v], rows_v, sem)
            pltpu.sync_copy(wexp_hbm.at[pl.ds(row0, RPC), :], wexp_v)
            cp.wait()

            def t_body(t, carry):
                for cc in range(C // LANES):
                    acc = jnp.zeros((LANES,), jnp.float32)
                    for j in range(KNN):
                        r = t * KNN + j
                        acc = acc + wexp_v[r, :] * rows_v[r, pl.ds(cc * LANES, LANES)]
                    outb_v[t, pl.ds(cc * LANES, LANES)] = acc
                return carry

            lax.fori_loop(0, TPC, t_body, 0)
            pltpu.sync_copy(outb_v, out_hbm.at[pl.ds(wid * TPW + ch * TPC, TPC), :])

    return k(feat, idx_flat, wexp)


def kernel(source_points, source_features, target_points):
    spT8 = jnp.zeros((8, N), jnp.float32).at[:3, :].set(source_points.T)
    tgt8 = jnp.zeros((M, 8), jnp.float32).at[:, :3].set(target_points)
    knn_i, w = _topk_weights(tgt8, spT8)
    idx_flat = knn_i.reshape(M * KNN)
    wexp = jnp.broadcast_to(w.reshape(M * KNN, 1), (M * KNN, LANES))
    return _sc_gather(source_features, idx_flat, wexp)


# speedup vs baseline: 12.9454x; 1.2290x over previous
import functools

import jax
import jax.numpy as jnp
from jax import lax
from jax.experimental import pallas as pl
from jax.experimental.pallas import tpu as pltpu
from jax.experimental.pallas import tpu_sc as plsc

M = 4096
N = 4096
C = 256
KNN = 8
TBLK = 256
GRID = M // TBLK


NCHK = 8
CW = N // NCHK


def _tc_topk_body(tgt_ref, spT_ref, idx_ref, w_ref):
    tb = tgt_ref[...]
    dist = jnp.zeros((TBLK, N), jnp.float32)
    for d in range(3):
        diff = tb[:, d:d + 1] - spT_ref[d:d + 1, :]
        dist = dist + diff * diff
    iota = lax.broadcasted_iota(jnp.int32, (TBLK, N), 1)
    bits = lax.bitcast_convert_type(dist + 1.0, jnp.int32)
    tags = lax.shift_right_logical(iota, 9)
    T = lax.bitcast_convert_type((bits & (-8)) | tags, jnp.float32)
    iota512f = lax.broadcasted_iota(jnp.int32, (TBLK, CW), 1).astype(jnp.float32)
    vals, gidxs = [], []
    for p in range(KNN):
        f = T[:, 0:CW]
        for c in range(1, NCHK):
            f = jnp.minimum(f, T[:, c * CW:(c + 1) * CW])
        mf = jnp.min(f, axis=1, keepdims=True)
        cand = jnp.where(f == mf, iota512f, jnp.float32(1e9))
        mi = jnp.min(cand, axis=1, keepdims=True)
        mfb = lax.bitcast_convert_type(mf, jnp.int32)
        cstar = mfb & 7
        gidx = cstar * CW + mi.astype(jnp.int32)
        dval = lax.bitcast_convert_type(mfb & (-8), jnp.float32) - 1.0
        vals.append(dval)
        gidxs.append(gidx)
        if p < KNN - 1:
            T = jnp.where(iota == gidx, jnp.float32(jnp.inf), T)
    knn_d = jnp.concatenate(vals, axis=1)
    knn_i = jnp.concatenate(gidxs, axis=1)
    w = 1.0 / (knn_d + 1e-8)
    w = w / jnp.sum(w, axis=1, keepdims=True)
    idx_ref[...] = knn_i
    w_ref[...] = w


def _topk_weights(tgt8, spT8):
    return pl.pallas_call(
        _tc_topk_body,
        grid=(GRID,),
        in_specs=[
            pl.BlockSpec((TBLK, 8), lambda i: (i, 0)),
            pl.BlockSpec((8, N), lambda i: (0, 0)),
        ],
        out_specs=[
            pl.BlockSpec((TBLK, KNN), lambda i: (i, 0)),
            pl.BlockSpec((TBLK, KNN), lambda i: (i, 0)),
        ],
        out_shape=[
            jax.ShapeDtypeStruct((M, KNN), jnp.int32),
            jax.ShapeDtypeStruct((M, KNN), jnp.float32),
        ],
    )(tgt8, spT8)


NUM_SC = 2
NUM_SUBCORES = 16
NW = NUM_SC * NUM_SUBCORES
TPW = M // NW
RPC = 128
TPC = RPC // KNN
NCH = TPW // TPC
LANES = 16


def _sc_gather(feat, idx_flat, w_flat):
    mesh = plsc.VectorSubcoreMesh(core_axis_name="c", subcore_axis_name="s")
    rpw = TPW * KNN

    @functools.partial(
        pl.kernel,
        mesh=mesh,
        out_type=jax.ShapeDtypeStruct((M, C), jnp.float32),
        scratch_types=[
            pltpu.VMEM((rpw,), jnp.int32),
            pltpu.VMEM((rpw,), jnp.float32),
            pltpu.VMEM((2, RPC, C), jnp.float32),
            pltpu.VMEM((2, TPC, C), jnp.float32),
            pltpu.SemaphoreType.DMA,
            pltpu.SemaphoreType.DMA,
            pltpu.SemaphoreType.DMA,
            pltpu.SemaphoreType.DMA,
        ],
    )
    def k(feat_hbm, idxf_hbm, wf_hbm, out_hbm, idx_v, w_v, rows_v, outb_v,
          sg0, sg1, so0, so1):
        sg = (sg0, sg1)
        so = (so0, so1)
        cid = lax.axis_index("c")
        sid = lax.axis_index("s")
        wid = sid * NUM_SC + cid
        base = wid * rpw
        pltpu.sync_copy(idxf_hbm.at[pl.ds(base, rpw)], idx_v)
        pltpu.sync_copy(wf_hbm.at[pl.ds(base, rpw)], w_v)
        gather_cp = [None, None]
        gather_cp[0] = pltpu.async_copy(feat_hbm.at[idx_v.at[pl.ds(0, RPC)]],
                                        rows_v.at[0], sg[0])
        out_cp = [None, None]
        for ch in range(NCH):
            b = ch % 2
            if ch + 1 < NCH:
                gather_cp[1 - b] = pltpu.async_copy(
                    feat_hbm.at[idx_v.at[pl.ds((ch + 1) * RPC, RPC)]],
                    rows_v.at[1 - b], sg[1 - b])
            gather_cp[b].wait()
            if out_cp[b] is not None:
                out_cp[b].wait()

            def pair_body(tt, carry):
                wpair = w_v[pl.ds(ch * RPC + tt * 2 * KNN, 2 * KNN)]
                for half in range(2):
                    t = tt * 2 + half
                    for cc in range(C // LANES):
                        acc = jnp.zeros((LANES,), jnp.float32)
                        for j in range(KNN):
                            acc = acc + wpair[half * KNN + j] * rows_v[
                                b, t * KNN + j, pl.ds(cc * LANES, LANES)]
                        outb_v[b, t, pl.ds(cc * LANES, LANES)] = acc
                return carry

            lax.fori_loop(0, TPC // 2, pair_body, 0)
            out_cp[b] = pltpu.async_copy(
                outb_v.at[b], out_hbm.at[pl.ds(wid * TPW + ch * TPC, TPC), :],
                so[b])
        out_cp[0].wait()
        out_cp[1].wait()

    return k(feat, idx_flat, w_flat)


def kernel(source_points, source_features, target_points):
    spT8 = jnp.zeros((8, N), jnp.float32).at[:3, :].set(source_points.T)
    tgt8 = jnp.zeros((M, 8), jnp.float32).at[:, :3].set(target_points)
    knn_i, w = _topk_weights(tgt8, spT8)
    return _sc_gather(source_features, knn_i.reshape(M * KNN),
                      w.reshape(M * KNN))


# speedup vs baseline: 13.5716x; 1.0484x over previous
import functools

import jax
import jax.numpy as jnp
from jax import lax
from jax.experimental import pallas as pl
from jax.experimental.pallas import tpu as pltpu
from jax.experimental.pallas import tpu_sc as plsc

M = 4096
N = 4096
C = 256
KNN = 8
TBLK = 256
GRID = M // TBLK


NCHK = 8
CW = N // NCHK


def _tc_topk_body(tgt_ref, spT_ref, idx_ref, w_ref):
    tb = tgt_ref[...]
    dist = jnp.zeros((TBLK, N), jnp.float32)
    for d in range(3):
        diff = tb[:, d:d + 1] - spT_ref[d:d + 1, :]
        dist = dist + diff * diff
    iota = lax.broadcasted_iota(jnp.int32, (TBLK, N), 1)
    bits = lax.bitcast_convert_type(dist + 1.0, jnp.int32)
    tags = lax.shift_right_logical(iota, 9)
    T = lax.bitcast_convert_type((bits & (-8)) | tags, jnp.float32)
    iota512f = lax.broadcasted_iota(jnp.int32, (TBLK, CW), 1).astype(jnp.float32)
    vals, gidxs = [], []
    for p in range(KNN):
        f = T[:, 0:CW]
        for c in range(1, NCHK):
            f = jnp.minimum(f, T[:, c * CW:(c + 1) * CW])
        mf = jnp.min(f, axis=1, keepdims=True)
        cand = jnp.where(f == mf, iota512f, jnp.float32(1e9))
        mi = jnp.min(cand, axis=1, keepdims=True)
        mfb = lax.bitcast_convert_type(mf, jnp.int32)
        cstar = mfb & 7
        gidx = cstar * CW + mi.astype(jnp.int32)
        dval = lax.bitcast_convert_type(mfb & (-8), jnp.float32) - 1.0
        vals.append(dval)
        gidxs.append(gidx)
        if p < KNN - 1:
            T = jnp.where(iota == gidx, jnp.float32(jnp.inf), T)
    knn_d = jnp.concatenate(vals, axis=1)
    knn_i = jnp.concatenate(gidxs, axis=1)
    w = 1.0 / (knn_d + 1e-8)
    w = w / jnp.sum(w, axis=1, keepdims=True)
    idx_ref[...] = knn_i
    w_ref[...] = w


NGRP = 2
MG = M // NGRP


def _topk_weights(tgt8_grp, spT8):
    return pl.pallas_call(
        _tc_topk_body,
        grid=(MG // TBLK,),
        in_specs=[
            pl.BlockSpec((TBLK, 8), lambda i: (i, 0)),
            pl.BlockSpec((8, N), lambda i: (0, 0)),
        ],
        out_specs=[
            pl.BlockSpec((TBLK, KNN), lambda i: (i, 0)),
            pl.BlockSpec((TBLK, KNN), lambda i: (i, 0)),
        ],
        out_shape=[
            jax.ShapeDtypeStruct((MG, KNN), jnp.int32),
            jax.ShapeDtypeStruct((MG, KNN), jnp.float32),
        ],
    )(tgt8_grp, spT8)


NUM_SC = 2
NUM_SUBCORES = 16
NW = NUM_SC * NUM_SUBCORES
TPW = MG // NW
RPC = 128
TPC = RPC // KNN
NCH = TPW // TPC
LANES = 16


def _sc_gather(feat, idx_flat, w_flat):
    mesh = plsc.VectorSubcoreMesh(core_axis_name="c", subcore_axis_name="s")
    rpw = TPW * KNN

    @functools.partial(
        pl.kernel,
        mesh=mesh,
        out_type=jax.ShapeDtypeStruct((MG, C), jnp.float32),
        scratch_types=[
            pltpu.VMEM((rpw,), jnp.int32),
            pltpu.VMEM((rpw,), jnp.float32),
            pltpu.VMEM((2, RPC, C), jnp.float32),
            pltpu.VMEM((2, TPC, C), jnp.float32),
            pltpu.SemaphoreType.DMA,
            pltpu.SemaphoreType.DMA,
            pltpu.SemaphoreType.DMA,
            pltpu.SemaphoreType.DMA,
        ],
    )
    def k(feat_hbm, idxf_hbm, wf_hbm, out_hbm, idx_v, w_v, rows_v, outb_v,
          sg0, sg1, so0, so1):
        sg = (sg0, sg1)
        so = (so0, so1)
        cid = lax.axis_index("c")
        sid = lax.axis_index("s")
        wid = sid * NUM_SC + cid
        base = wid * rpw
        pltpu.sync_copy(idxf_hbm.at[pl.ds(base, rpw)], idx_v)
        pltpu.sync_copy(wf_hbm.at[pl.ds(base, rpw)], w_v)
        gather_cp = [None, None]
        gather_cp[0] = pltpu.async_copy(feat_hbm.at[idx_v.at[pl.ds(0, RPC)]],
                                        rows_v.at[0], sg[0])
        out_cp = [None, None]
        for ch in range(NCH):
            b = ch % 2
            if ch + 1 < NCH:
                gather_cp[1 - b] = pltpu.async_copy(
                    feat_hbm.at[idx_v.at[pl.ds((ch + 1) * RPC, RPC)]],
                    rows_v.at[1 - b], sg[1 - b])
            gather_cp[b].wait()
            if out_cp[b] is not None:
                out_cp[b].wait()

            def pair_body(tt, carry):
                wpair = w_v[pl.ds(ch * RPC + tt * 2 * KNN, 2 * KNN)]
                for half in range(2):
                    t = tt * 2 + half
                    for cc in range(C // LANES):
                        acc = jnp.zeros((LANES,), jnp.float32)
                        for j in range(KNN):
                            acc = acc + wpair[half * KNN + j] * rows_v[
                                b, t * KNN + j, pl.ds(cc * LANES, LANES)]
                        outb_v[b, t, pl.ds(cc * LANES, LANES)] = acc
                return carry

            lax.fori_loop(0, TPC // 2, pair_body, 0)
            out_cp[b] = pltpu.async_copy(
                outb_v.at[b], out_hbm.at[pl.ds(wid * TPW + ch * TPC, TPC), :],
                so[b])
        out_cp[0].wait()
        out_cp[1].wait()

    return k(feat, idx_flat, w_flat)


def kernel(source_points, source_features, target_points):
    spT8 = jnp.zeros((8, N), jnp.float32).at[:3, :].set(source_points.T)
    tgt8 = jnp.zeros((M, 8), jnp.float32).at[:, :3].set(target_points)
    outs = []
    for g in range(NGRP):
        knn_i, w = _topk_weights(
            lax.slice_in_dim(tgt8, g * MG, (g + 1) * MG, axis=0), spT8)
        outs.append(_sc_gather(source_features, knn_i.reshape(MG * KNN),
                               w.reshape(MG * KNN)))
    return jnp.concatenate(outs, axis=0)
